# baseline (speedup 1.0000x reference)
import jax
import jax.numpy as jnp
from jax import lax
from jax.experimental import pallas as pl
from jax.experimental.pallas import tpu as pltpu
from jax.experimental.pallas import tpu_sc as plsc

L = 200
B = 1024
M = 64
BLK = 128
BPR = B // BLK
NROWS_STAGE = 8
NLANES = 16


def _body(W_emb, W_re, pos_emb, ly, lp, ry, re, zeros_hbm,
          out_l, out_p, out_r, out_e, idx_v, rows_v, tbuf, sem_g, sem_w):
    cid = lax.axis_index("c")
    sid = lax.axis_index("s")
    wid = sid * 2 + cid
    t0 = jnp.where(wid < 24, 6 * wid, 7 * wid - 24)
    nblk = jnp.where(wid < 24, 6 * BPR, 7 * BPR)
    start = jnp.where(wid == 0, BPR, 0)
    rl_lp = jnp.clip(t0 - 1, 0, L - NROWS_STAGE)
    dr_lp = t0 - 1 - rl_lp
    rl_re = jnp.minimum(t0, L - NROWS_STAGE)
    dr_re = t0 - rl_re

    iota = lax.iota(jnp.int32, NLANES)

    def transpose_block(b):
        def tr_body(j, carry):
            cidx = iota * 0 + j
            for g in range(BLK // NLANES):
                ridx = iota + (g * NLANES)
                v = plsc.load_gather(rows_v.at[b], [ridx, cidx])
                tbuf.at[b][j, pl.ds(g * NLANES, NLANES)] = v
            return carry
        lax.fori_loop(0, M, tr_body, 0)

    for k, (table, idx, out, rl, dr) in enumerate(
        ((W_emb, ly, out_l, rl_lp, dr_lp),
         (pos_emb, lp, out_p, rl_lp, dr_lp),
         (W_emb, ry, out_r, rl_re, dr_re),
         (W_re, re, out_e, rl_re, dr_re))):
        pltpu.sync_copy(idx.at[pl.ds(rl, NROWS_STAGE)], idx_v.at[k])

        def fire_gather(j, b, table=table, k=k, dr=dr):
            r = lax.div(j, BPR) + dr
            c = lax.rem(j, BPR) * BLK
            pltpu.async_copy(
                table.at[idx_v.at[k].at[r].at[pl.ds(c, BLK)]],
                rows_v.at[b], sem_g)

        def fire_write(j, b, out=out):
            t = t0 + lax.div(j, BPR)
            c = lax.rem(j, BPR) * BLK
            pltpu.async_copy(
                tbuf.at[b], out.at[t].at[:, pl.ds(c, BLK)], sem_w)

        def drain_gather(b, out=out):
            pltpu.make_async_copy(out.at[0].at[:, pl.ds(0, BLK)],
                                  rows_v.at[b], sem_g).wait()

        def drain_write(b, out=out):
            pltpu.make_async_copy(tbuf.at[b],
                                  out.at[0].at[:, pl.ds(0, BLK)], sem_w).wait()

        fire_gather(start, start % 2)

        def body(j, carry):
            b = lax.rem(j, 2)

            @pl.when(j >= start + 1)
            def _():
                drain_write(lax.rem(j - 1, 2))

            @pl.when(j + 1 < nblk)
            def _():
                fire_gather(j + 1, lax.rem(j + 1, 2))

            drain_gather(b)
            transpose_block(b)
            fire_write(j, b)
            return carry

        lax.fori_loop(start, nblk, body, 0)
        drain_write(lax.rem(nblk - 1, 2))

    @pl.when(wid == 0)
    def _():
        for out in (out_l, out_p, out_r, out_e):
            pltpu.sync_copy(zeros_hbm, out.at[0])


@jax.jit
def kernel(ly, lp, ry, re, W_emb, W_re, pos_emb):
    zeros_hbm = jnp.zeros((M, B), jnp.float32)
    mesh = plsc.VectorSubcoreMesh(core_axis_name="c", subcore_axis_name="s")
    f = pl.kernel(
        _body,
        out_type=[jax.ShapeDtypeStruct((L, M, B), jnp.float32)] * 4,
        mesh=mesh,
        scratch_types=[
            pltpu.VMEM((4, NROWS_STAGE, B), jnp.int32),
            pltpu.VMEM((2, BLK, M), jnp.float32),
            pltpu.VMEM((2, M, BLK), jnp.float32),
            pltpu.SemaphoreType.DMA,
            pltpu.SemaphoreType.DMA,
        ],
        compiler_params=pltpu.CompilerParams(
            use_tc_tiling_on_sc=False, needs_layout_passes=False),
    )
    outs = f(W_emb, W_re, pos_emb,
             ly.astype(jnp.int32), lp.astype(jnp.int32),
             ry.astype(jnp.int32), re.astype(jnp.int32), zeros_hbm)
    return tuple(jnp.swapaxes(o, 1, 2) for o in outs)


# speedup vs baseline: 1.8157x; 1.8157x over previous
import functools

import jax
import jax.numpy as jnp
from jax import lax
from jax.experimental import pallas as pl
from jax.experimental.pallas import tpu as pltpu
from jax.experimental.pallas import tpu_sc as plsc

L = 200
B = 1024
M = 64
BLK = 128
BPR = B // BLK
NROWS_STAGE = 8


def _make_body(shifted):
    def _body(table, idx, zeros_hbm, out, idx_v, rows_v, sem_g, sem_w):
        cid = lax.axis_index("c")
        sid = lax.axis_index("s")
        wid = sid * 2 + cid
        t0 = jnp.where(wid < 24, 6 * wid, 7 * wid - 24)
        nblk = jnp.where(wid < 24, 6 * BPR, 7 * BPR)
        start = jnp.where(wid == 0, BPR, 0)
        if shifted:
            rl = jnp.clip(t0 - 1, 0, L - NROWS_STAGE)
            dr = t0 - 1 - rl
        else:
            rl = jnp.minimum(t0, L - NROWS_STAGE)
            dr = t0 - rl

        pltpu.sync_copy(idx.at[pl.ds(rl, NROWS_STAGE)], idx_v)

        def fire_gather(j, b):
            r = lax.div(j, BPR) + dr
            c = lax.rem(j, BPR) * BLK
            pltpu.async_copy(
                table.at[idx_v.at[r].at[pl.ds(c, BLK)]], rows_v.at[b], sem_g)

        def fire_write(j, b):
            t = t0 + lax.div(j, BPR)
            c = lax.rem(j, BPR) * BLK
            pltpu.async_copy(
                rows_v.at[b], out.at[t].at[pl.ds(c, BLK)], sem_w)

        def drain_gather(b):
            pltpu.make_async_copy(out.at[0].at[pl.ds(0, BLK)],
                                  rows_v.at[b], sem_g).wait()

        def drain_write(b):
            pltpu.make_async_copy(rows_v.at[b],
                                  out.at[0].at[pl.ds(0, BLK)], sem_w).wait()

        fire_gather(start, start % 2)

        def body(j, carry):
            b = lax.rem(j, 2)

            @pl.when(j >= start + 1)
            def _():
                drain_write(lax.rem(j - 1, 2))

            @pl.when(j + 1 < nblk)
            def _():
                fire_gather(j + 1, lax.rem(j + 1, 2))

            drain_gather(b)
            fire_write(j, b)
            return carry

        lax.fori_loop(start, nblk, body, 0)
        drain_write(lax.rem(nblk - 1, 2))

        @pl.when(wid == 0)
        def _():
            pltpu.sync_copy(zeros_hbm, out.at[0])

    return _body


def _gather_call(table, idx, zeros_hbm, shifted):
    mesh = plsc.VectorSubcoreMesh(core_axis_name="c", subcore_axis_name="s")
    f = pl.kernel(
        _make_body(shifted),
        out_type=jax.ShapeDtypeStruct((L, B, M), jnp.float32),
        mesh=mesh,
        scratch_types=[
            pltpu.VMEM((NROWS_STAGE, B), jnp.int32),
            pltpu.VMEM((2, BLK, M), jnp.float32),
            pltpu.SemaphoreType.DMA,
            pltpu.SemaphoreType.DMA,
        ],
        compiler_params=pltpu.CompilerParams(use_tc_tiling_on_sc=False),
    )
    return f(table, idx, zeros_hbm)


@jax.jit
def kernel(ly, lp, ry, re, W_emb, W_re, pos_emb):
    zeros_hbm = jnp.zeros((B, M), jnp.float32)
    out_e = _gather_call(W_re, re.astype(jnp.int32), zeros_hbm, shifted=False)
    out_p = _gather_call(pos_emb, lp.astype(jnp.int32), zeros_hbm, shifted=True)
    out_l = _gather_call(W_emb, ly.astype(jnp.int32), zeros_hbm, shifted=True)
    out_r = _gather_call(W_emb, ry.astype(jnp.int32), zeros_hbm, shifted=False)
    return (out_l, out_p, out_r, out_e)


# speedup vs baseline: 2.1680x; 1.1940x over previous
import functools

import jax
import jax.numpy as jnp
from jax import lax
from jax.experimental import pallas as pl
from jax.experimental.pallas import tpu as pltpu
from jax.experimental.pallas import tpu_sc as plsc

L = 200
B = 1024
M = 64
BLK = 128
BPR = B // BLK
NROWS_STAGE = 8


def _make_body(shifted):
    def _body(table, idx, zeros_hbm, out, idx_v, rows_v, sem_g, sem_w):
        cid = lax.axis_index("c")
        sid = lax.axis_index("s")
        wid = sid * 2 + cid
        t0 = jnp.where(wid < 24, 6 * wid, 7 * wid - 24)
        nblk = jnp.where(wid < 24, 6 * BPR, 7 * BPR)
        start = jnp.where(wid == 0, BPR, 0)
        if shifted:
            rl = jnp.clip(t0 - 1, 0, L - NROWS_STAGE)
            dr = t0 - 1 - rl
        else:
            rl = jnp.minimum(t0, L - NROWS_STAGE)
            dr = t0 - rl

        pltpu.sync_copy(idx.at[pl.ds(rl, NROWS_STAGE)], idx_v)

        def fire_gather(j, b):
            r = lax.div(j, BPR) + dr
            c = lax.rem(j, BPR) * BLK
            pltpu.async_copy(
                table.at[idx_v.at[r].at[pl.ds(c, BLK)]], rows_v.at[b], sem_g)

        def fire_write(j, b):
            t = t0 + lax.div(j, BPR)
            c = lax.rem(j, BPR) * BLK
            pltpu.async_copy(
                rows_v.at[b], out.at[t].at[pl.ds(c, BLK), pl.ds(0, M)], sem_w)

        def drain_gather(b):
            pltpu.make_async_copy(out.at[0].at[pl.ds(0, BLK), pl.ds(0, M)],
                                  rows_v.at[b], sem_g).wait()

        def drain_write(b):
            pltpu.make_async_copy(rows_v.at[b],
                                  out.at[0].at[pl.ds(0, BLK), pl.ds(0, M)], sem_w).wait()

        fire_gather(start, start % 2)

        def body(j, carry):
            b = lax.rem(j, 2)

            @pl.when(j >= start + 1)
            def _():
                drain_write(lax.rem(j - 1, 2))

            @pl.when(j + 1 < nblk)
            def _():
                fire_gather(j + 1, lax.rem(j + 1, 2))

            drain_gather(b)
            fire_write(j, b)
            return carry

        lax.fori_loop(start, nblk, body, 0)
        drain_write(lax.rem(nblk - 1, 2))

        @pl.when(wid == 0)
        def _():
            pltpu.sync_copy(zeros_hbm, out.at[0].at[:, pl.ds(0, M)])

    return _body


def _gather_call(table, idx, zeros_hbm, shifted):
    mesh = plsc.VectorSubcoreMesh(core_axis_name="c", subcore_axis_name="s")
    f = pl.kernel(
        _make_body(shifted),
        out_type=jax.ShapeDtypeStruct((L, B, 2 * M), jnp.float32),
        mesh=mesh,
        scratch_types=[
            pltpu.VMEM((NROWS_STAGE, B), jnp.int32),
            pltpu.VMEM((2, BLK, M), jnp.float32),
            pltpu.SemaphoreType.DMA,
            pltpu.SemaphoreType.DMA,
        ],
        compiler_params=pltpu.CompilerParams(use_tc_tiling_on_sc=False),
    )
    return f(table, idx, zeros_hbm)[:, :, :M]


@jax.jit
def kernel(ly, lp, ry, re, W_emb, W_re, pos_emb):
    zeros_hbm = jnp.zeros((B, M), jnp.float32)
    out_e = _gather_call(W_re, re.astype(jnp.int32), zeros_hbm, shifted=False)
    out_p = _gather_call(pos_emb, lp.astype(jnp.int32), zeros_hbm, shifted=True)
    out_l = _gather_call(W_emb, ly.astype(jnp.int32), zeros_hbm, shifted=True)
    out_r = _gather_call(W_emb, ry.astype(jnp.int32), zeros_hbm, shifted=False)
    return (out_l, out_p, out_r, out_e)


# speedup vs baseline: 2.2683x; 1.0463x over previous
import jax
import jax.numpy as jnp
from jax import lax
from jax.experimental import pallas as pl
from jax.experimental.pallas import tpu as pltpu
from jax.experimental.pallas import tpu_sc as plsc

L = 200
B = 1024
M = 64
MP = 2 * M
BLK = 128
BPR = B // BLK
NROWS_STAGE = 8


def _make_body(shifts):
    def _body(table1, idx1, table2, idx2, zeros_hbm, out1, out2,
              idx_v, rows_v, sem_g, sem_w):
        cid = lax.axis_index("c")
        sid = lax.axis_index("s")
        wid = sid * 2 + cid
        t0 = jnp.where(wid < 24, 6 * wid, 7 * wid - 24)
        nblk = jnp.where(wid < 24, 6 * BPR, 7 * BPR)
        start = jnp.where(wid == 0, BPR, 0)

        for k, (table, idx, out, shifted) in enumerate(
            ((table1, idx1, out1, shifts[0]), (table2, idx2, out2, shifts[1]))):
            if shifted:
                rl = jnp.clip(t0 - 1, 0, L - NROWS_STAGE)
                dr = t0 - 1 - rl
            else:
                rl = jnp.minimum(t0, L - NROWS_STAGE)
                dr = t0 - rl

            pltpu.sync_copy(idx.at[pl.ds(rl, NROWS_STAGE)], idx_v.at[k])

            def fire_gather(j, b, table=table, k=k, dr=dr):
                r = lax.div(j, BPR) + dr
                c = lax.rem(j, BPR) * BLK
                pltpu.async_copy(
                    table.at[idx_v.at[k].at[r].at[pl.ds(c, BLK)]],
                    rows_v.at[b], sem_g)

            def fire_write(j, b, out=out):
                t = t0 + lax.div(j, BPR)
                c = lax.rem(j, BPR) * BLK
                pltpu.async_copy(
                    rows_v.at[b], out.at[t].at[pl.ds(c, BLK), pl.ds(0, M)],
                    sem_w)

            def drain_gather(b, out=out):
                pltpu.make_async_copy(
                    out.at[0].at[pl.ds(0, BLK), pl.ds(0, M)],
                    rows_v.at[b], sem_g).wait()

            def drain_write(b, out=out):
                pltpu.make_async_copy(
                    rows_v.at[b],
                    out.at[0].at[pl.ds(0, BLK), pl.ds(0, M)], sem_w).wait()

            fire_gather(start, start % 2)

            def body(j, carry):
                b = lax.rem(j, 2)

                @pl.when(j >= start + 1)
                def _():
                    drain_write(lax.rem(j - 1, 2))

                @pl.when(j + 1 < nblk)
                def _():
                    fire_gather(j + 1, lax.rem(j + 1, 2))

                drain_gather(b)
                fire_write(j, b)
                return carry

            lax.fori_loop(start, nblk, body, 0)
            drain_write(lax.rem(nblk - 1, 2))

        @pl.when(wid == 0)
        def _():
            for out in (out1, out2):
                pltpu.sync_copy(zeros_hbm, out.at[0].at[:, pl.ds(0, M)])

    return _body


def _gather_call2(table1, idx1, table2, idx2, zeros_hbm, shifts):
    mesh = plsc.VectorSubcoreMesh(core_axis_name="c", subcore_axis_name="s")
    f = pl.kernel(
        _make_body(shifts),
        out_type=[jax.ShapeDtypeStruct((L, B, MP), jnp.float32)] * 2,
        mesh=mesh,
        scratch_types=[
            pltpu.VMEM((2, NROWS_STAGE, B), jnp.int32),
            pltpu.VMEM((2, BLK, M), jnp.float32),
            pltpu.SemaphoreType.DMA,
            pltpu.SemaphoreType.DMA,
        ],
        compiler_params=pltpu.CompilerParams(use_tc_tiling_on_sc=False),
    )
    o1, o2 = f(table1, idx1, table2, idx2, zeros_hbm)
    return o1[:, :, :M], o2[:, :, :M]


@jax.jit
def kernel(ly, lp, ry, re, W_emb, W_re, pos_emb):
    zeros_hbm = jnp.zeros((B, M), jnp.float32)
    out_e, out_p = _gather_call2(
        W_re, re.astype(jnp.int32), pos_emb, lp.astype(jnp.int32),
        zeros_hbm, (False, True))
    out_l, out_r = _gather_call2(
        W_emb, ly.astype(jnp.int32), W_emb, ry.astype(jnp.int32),
        zeros_hbm, (True, False))
    return (out_l, out_p, out_r, out_e)
